# baseline (speedup 1.0000x reference)
import functools

import jax
import jax.numpy as jnp
from jax import lax
from jax.experimental import pallas as pl
from jax.experimental.pallas import tpu as pltpu
from jax.experimental.pallas import tpu_sc as plsc

_OFFSET = 2


def kernel(input_ids, pos_table):
    bsz, seq_len = input_ids.shape
    dim = pos_table.shape[-1]

    info = plsc.get_sparse_core_info()
    num_cores, num_subcores = info.num_cores, info.num_subcores
    num_workers = num_cores * num_subcores
    rows_per_worker = seq_len // num_workers
    chunk_rows = 32
    n_chunks = rows_per_worker // chunk_rows

    @functools.partial(
        pl.kernel,
        mesh=plsc.VectorSubcoreMesh(core_axis_name="c", subcore_axis_name="s"),
        out_type=jax.ShapeDtypeStruct((bsz, seq_len, dim), jnp.float32),
        scratch_types=[
            pltpu.VMEM((chunk_rows, dim), jnp.float32),
            pltpu.VMEM((chunk_rows, dim), jnp.float32),
            pltpu.SemaphoreType.DMA,
            pltpu.SemaphoreType.DMA,
        ],
        compiler_params=pltpu.CompilerParams(use_tc_tiling_on_sc=False),
    )
    def pe_kernel(table_hbm, out_hbm, buf0, buf1, in_sem, out_sem):
        wid = lax.axis_index("s") * num_cores + lax.axis_index("c")
        base = wid * rows_per_worker
        bufs = (buf0, buf1)

        def read(g, buf):
            row0 = base + g * chunk_rows
            return pltpu.async_copy(
                table_hbm.at[pl.ds(row0 + _OFFSET, chunk_rows), :], buf, in_sem
            )

        def writes(g, buf):
            row0 = base + g * chunk_rows
            return [
                pltpu.async_copy(
                    buf, out_hbm.at[b, pl.ds(row0, chunk_rows), :], out_sem
                )
                for b in range(bsz)
            ]

        rd = read(0, bufs[0])
        pending = None
        for g in range(n_chunks):
            rd.wait()
            if pending is not None:
                for c in pending:
                    c.wait()
            if g + 1 < n_chunks:
                rd = read(g + 1, bufs[(g + 1) % 2])
            pending = writes(g, bufs[g % 2])
        for c in pending:
            c.wait()

    return pe_kernel(pos_table)


# speedup vs baseline: 2.0139x; 2.0139x over previous
import functools

import jax
import jax.numpy as jnp
from jax import lax
from jax.experimental import pallas as pl
from jax.experimental.pallas import tpu as pltpu
from jax.experimental.pallas import tpu_sc as plsc

_OFFSET = 2


def kernel(input_ids, pos_table):
    bsz, seq_len = input_ids.shape
    dim = pos_table.shape[-1]

    info = plsc.get_sparse_core_info()
    num_cores, num_subcores = info.num_cores, info.num_subcores
    num_lanes = info.num_lanes
    num_workers = num_cores * num_subcores
    rows_per_worker = seq_len // num_workers
    chunk_rows = 32
    n_chunks = rows_per_worker // chunk_rows

    read_rows = chunk_rows + 8

    @functools.partial(
        pl.kernel,
        mesh=plsc.VectorSubcoreMesh(core_axis_name="c", subcore_axis_name="s"),
        out_type=jax.ShapeDtypeStruct((bsz, seq_len, dim), jnp.float32),
        scratch_types=[
            pltpu.VMEM((read_rows, dim), jnp.float32),
            pltpu.VMEM((read_rows, dim), jnp.float32),
            pltpu.SemaphoreType.DMA,
            pltpu.SemaphoreType.DMA,
        ],
    )
    def pe_kernel(table_hbm, out_hbm, buf0, buf1, in_sem, out_sem):
        wid = lax.axis_index("s") * num_cores + lax.axis_index("c")
        base = wid * rows_per_worker
        bufs = (buf0, buf1)

        def read(g, buf):
            row0 = base + g * chunk_rows
            return pltpu.async_copy(
                table_hbm.at[pl.ds(row0, read_rows), :], buf, in_sem
            )

        def shift(buf):
            def row_body(r, _):
                for j in range(dim // num_lanes):
                    c = j * num_lanes
                    buf[r, pl.ds(c, num_lanes)] = buf[r + _OFFSET, pl.ds(c, num_lanes)]
                return 0

            lax.fori_loop(0, chunk_rows, row_body, 0)

        def writes(g, buf):
            row0 = base + g * chunk_rows
            return [
                pltpu.async_copy(
                    buf.at[pl.ds(0, chunk_rows), :],
                    out_hbm.at[b, pl.ds(row0, chunk_rows), :],
                    out_sem,
                )
                for b in range(bsz)
            ]

        rd = read(0, bufs[0])
        pending = None
        for g in range(n_chunks):
            rd.wait()
            if pending is not None:
                for c in pending:
                    c.wait()
            if g + 1 < n_chunks:
                rd = read(g + 1, bufs[(g + 1) % 2])
            shift(bufs[g % 2])
            pending = writes(g, bufs[g % 2])
        for c in pending:
            c.wait()

    return pe_kernel(pos_table)


# speedup vs baseline: 2.7761x; 1.3785x over previous
import functools

import jax
import jax.numpy as jnp
from jax import lax
from jax.experimental import pallas as pl
from jax.experimental.pallas import tpu as pltpu
from jax.experimental.pallas import tpu_sc as plsc

_OFFSET = 2


def kernel(input_ids, pos_table):
    bsz, seq_len = input_ids.shape
    dim = pos_table.shape[-1]

    info = plsc.get_sparse_core_info()
    num_cores, num_subcores = info.num_cores, info.num_subcores
    num_lanes = info.num_lanes
    num_workers = num_cores * num_subcores
    rows_per_worker = seq_len // num_workers
    chunk_rows = 32
    n_chunks = rows_per_worker // chunk_rows

    read_rows = chunk_rows + 8

    @functools.partial(
        pl.kernel,
        mesh=plsc.VectorSubcoreMesh(core_axis_name="c", subcore_axis_name="s"),
        out_type=jax.ShapeDtypeStruct((bsz, seq_len, dim), jnp.float32),
        scratch_types=[
            pltpu.VMEM((read_rows, dim), jnp.float32),
            pltpu.VMEM((read_rows, dim), jnp.float32),
            pltpu.VMEM((read_rows, dim), jnp.float32),
            pltpu.SemaphoreType.DMA,
            pltpu.SemaphoreType.DMA,
        ],
    )
    def pe_kernel(table_hbm, out_hbm, buf0, buf1, buf2, in_sem, out_sem):
        wid = lax.axis_index("s") * num_cores + lax.axis_index("c")
        base = wid * rows_per_worker
        bufs = (buf0, buf1, buf2)

        def read(g, buf):
            row0 = base + g * chunk_rows
            return pltpu.async_copy(
                table_hbm.at[pl.ds(row0, read_rows), :], buf, in_sem
            )

        def shift(buf):
            def row_body(r, _):
                for j in range(dim // num_lanes):
                    c = j * num_lanes
                    buf[r, pl.ds(c, num_lanes)] = buf[r + _OFFSET, pl.ds(c, num_lanes)]
                return 0

            lax.fori_loop(0, chunk_rows, row_body, 0)

        def writes(g, buf):
            row0 = base + g * chunk_rows
            return [
                pltpu.async_copy(
                    buf.at[pl.ds(0, chunk_rows), :],
                    out_hbm.at[b, pl.ds(row0, chunk_rows), :],
                    out_sem,
                )
                for b in range(bsz)
            ]

        rds = {0: read(0, bufs[0])}
        if n_chunks > 1:
            rds[1] = read(1, bufs[1])
        pending = {}
        for g in range(n_chunks):
            rds[g].wait()
            shift(bufs[g % 3])
            if g - 1 in pending:
                for c in pending.pop(g - 1):
                    c.wait()
            if g + 2 < n_chunks:
                rds[g + 2] = read(g + 2, bufs[(g + 2) % 3])
            pending[g] = writes(g, bufs[g % 3])
        for copies in pending.values():
            for c in copies:
                c.wait()

    return pe_kernel(pos_table)


# speedup vs baseline: 2.8307x; 1.0197x over previous
import functools

import jax
import jax.numpy as jnp
from jax import lax
from jax.experimental import pallas as pl
from jax.experimental.pallas import tpu as pltpu
from jax.experimental.pallas import tpu_sc as plsc

_OFFSET = 2


def kernel(input_ids, pos_table):
    bsz, seq_len = input_ids.shape
    dim = pos_table.shape[-1]

    info = plsc.get_sparse_core_info()
    num_cores, num_subcores = info.num_cores, info.num_subcores
    num_lanes = info.num_lanes
    num_workers = num_cores * num_subcores
    rows_per_worker = seq_len // num_workers
    chunk_rows = 32
    n_chunks = rows_per_worker // chunk_rows

    read_rows = chunk_rows + 8

    @functools.partial(
        pl.kernel,
        mesh=plsc.VectorSubcoreMesh(core_axis_name="c", subcore_axis_name="s"),
        out_type=jax.ShapeDtypeStruct((bsz, seq_len, dim), jnp.float32),
        scratch_types=[
            pltpu.VMEM((read_rows, dim), jnp.float32),
            pltpu.VMEM((read_rows, dim), jnp.float32),
            pltpu.VMEM((read_rows, dim), jnp.float32),
            pltpu.SemaphoreType.DMA,
            pltpu.SemaphoreType.DMA,
        ],
    )
    def pe_kernel(table_hbm, out_hbm, buf0, buf1, buf2, in_sem, out_sem):
        wid = lax.axis_index("s") * num_cores + lax.axis_index("c")
        base = wid * rows_per_worker
        bufs = (buf0, buf1, buf2)

        def read(g, buf):
            row0 = base + g * chunk_rows
            return pltpu.async_copy(
                table_hbm.at[pl.ds(row0, read_rows), :], buf, in_sem
            )

        def shift(buf):
            def group_body(k, _):
                r0 = pl.multiple_of(k * 8, 8)
                for i in range(8):
                    for j in range(dim // num_lanes):
                        c = j * num_lanes
                        buf[r0 + i, pl.ds(c, num_lanes)] = buf[
                            r0 + (i + _OFFSET), pl.ds(c, num_lanes)
                        ]
                return 0

            lax.fori_loop(0, chunk_rows // 8, group_body, 0)

        def writes(g, buf):
            row0 = base + g * chunk_rows
            return [
                pltpu.async_copy(
                    buf.at[pl.ds(0, chunk_rows), :],
                    out_hbm.at[b, pl.ds(row0, chunk_rows), :],
                    out_sem,
                )
                for b in range(bsz)
            ]

        rds = {0: read(0, bufs[0])}
        if n_chunks > 1:
            rds[1] = read(1, bufs[1])
        pending = {}
        for g in range(n_chunks):
            rds[g].wait()
            shift(bufs[g % 3])
            if g - 1 in pending:
                for c in pending.pop(g - 1):
                    c.wait()
            if g + 2 < n_chunks:
                rds[g + 2] = read(g + 2, bufs[(g + 2) % 3])
            pending[g] = writes(g, bufs[g % 3])
        for copies in pending.values():
            for c in copies:
                c.wait()

    return pe_kernel(pos_table)


# speedup vs baseline: 2.8401x; 1.0033x over previous
import functools

import jax
import jax.numpy as jnp
from jax import lax
from jax.experimental import pallas as pl
from jax.experimental.pallas import tpu as pltpu
from jax.experimental.pallas import tpu_sc as plsc

_OFFSET = 2


def kernel(input_ids, pos_table):
    bsz, seq_len = input_ids.shape
    dim = pos_table.shape[-1]

    info = plsc.get_sparse_core_info()
    num_cores, num_subcores = info.num_cores, info.num_subcores
    num_lanes = info.num_lanes
    num_workers = num_cores * num_subcores
    rows_per_worker = seq_len // num_workers
    chunk_rows = 32
    n_chunks = rows_per_worker // chunk_rows

    read_rows = chunk_rows + 8

    @functools.partial(
        pl.kernel,
        mesh=plsc.VectorSubcoreMesh(core_axis_name="c", subcore_axis_name="s"),
        out_type=jax.ShapeDtypeStruct((bsz, seq_len, dim), jnp.float32),
        scratch_types=[
            pltpu.VMEM((read_rows, dim), jnp.float32),
            pltpu.VMEM((read_rows, dim), jnp.float32),
            pltpu.VMEM((read_rows, dim), jnp.float32),
            pltpu.SemaphoreType.DMA,
            pltpu.SemaphoreType.DMA,
        ],
    )
    def pe_kernel(table_hbm, out_hbm, buf0, buf1, buf2, in_sem, out_sem):
        wid = lax.axis_index("s") * num_cores + lax.axis_index("c")
        base = wid * rows_per_worker
        bufs = (buf0, buf1, buf2)

        def read(g, buf):
            row0 = base + g * chunk_rows
            return pltpu.async_copy(
                table_hbm.at[pl.ds(row0, read_rows), :], buf, in_sem
            )

        def shift(buf):
            def group_body(k, _):
                r0 = pl.multiple_of(k * 8, 8)
                for i in range(8):
                    for j in range(dim // num_lanes):
                        c = j * num_lanes
                        buf[r0 + i, pl.ds(c, num_lanes)] = buf[
                            r0 + (i + _OFFSET), pl.ds(c, num_lanes)
                        ]
                return 0

            lax.fori_loop(0, chunk_rows // 8, group_body, 0)

        def writes(g, buf):
            row0 = base + g * chunk_rows
            return [
                pltpu.async_copy(
                    buf.at[pl.ds(0, chunk_rows), :],
                    out_hbm.at[b, pl.ds(row0, chunk_rows), :],
                    out_sem,
                )
                for b in range(bsz)
            ]

        rds = {0: read(0, bufs[0])}
        if n_chunks > 1:
            rds[1] = read(1, bufs[1])
        pending = {}
        for g in range(n_chunks):
            rds[g].wait()
            shift(bufs[g % 3])
            pending[g] = writes(g, bufs[g % 3])
            if g - 1 in pending:
                for c in pending.pop(g - 1):
                    c.wait()
            if g + 2 < n_chunks:
                rds[g + 2] = read(g + 2, bufs[(g + 2) % 3])
        for copies in pending.values():
            for c in copies:
                c.wait()

    return pe_kernel(pos_table)
